# baseline (speedup 1.0000x reference)
import jax
import jax.numpy as jnp
from jax.experimental import pallas as pl
from jax.experimental.pallas import tpu as pltpu

_N, _D, _H, _B = 16384, 512, 256, 16
_BLK = 2048
_NB = _N // _BLK
_DA = _D + 128


def _pool_body(xb, bb, w1t, w2t, out, acc, mprev, aprev):
    i = pl.program_id(0)

    @pl.when(i == 0)
    def _init():
        acc[:] = jnp.zeros_like(acc)

    x_bf = xb[:].astype(jnp.bfloat16)
    t_bf = jnp.tanh(jnp.dot(x_bf, w1t[:],
                            preferred_element_type=jnp.float32)
                    ).astype(jnp.bfloat16)
    a = jnp.dot(t_bf, w2t[:],
                preferred_element_type=jnp.float32)
    e_bf = jnp.exp(a).astype(jnp.bfloat16)
    b = bb[0]
    seg = jax.lax.broadcasted_iota(jnp.int32, (_B, _BLK), 0)
    maskf = (b == seg).astype(jnp.bfloat16)

    @pl.when(i > 0)
    def _pool_prev():
        acc[:] = acc[:] + jax.lax.dot_general(
            mprev[:], aprev[:], (((1,), (0,)), ((), ())),
            preferred_element_type=jnp.float32)

    mprev[:] = maskf
    aprev[:, :_D] = x_bf * e_bf
    aprev[:, _D:] = jnp.broadcast_to(e_bf, (_BLK, 128))

    @pl.when(i == _NB - 1)
    def _fin():
        z = acc[:] + jax.lax.dot_general(
            maskf, aprev[:], (((1,), (0,)), ((), ())),
            preferred_element_type=jnp.float32)
        out[:] = z[:, :_D] / (z[:, _D:_D + 1] + 1e-16)


def kernel(x, batch, W1, W2):
    batch3 = batch.astype(jnp.int32).reshape(_NB, 1, _BLK)
    w1t = W1.T.astype(jnp.bfloat16)
    w2t = W2.T.astype(jnp.bfloat16)
    return pl.pallas_call(
        _pool_body,
        grid=(_NB,),
        in_specs=[
            pl.BlockSpec((_BLK, _D), lambda i: (i, 0)),
            pl.BlockSpec((1, 1, _BLK), lambda i: (i, 0, 0)),
            pl.BlockSpec((_D, _H), lambda i: (0, 0)),
            pl.BlockSpec((_H, 1), lambda i: (0, 0)),
        ],
        out_specs=pl.BlockSpec((_B, _D), lambda i: (0, 0)),
        out_shape=jax.ShapeDtypeStruct((_B, _D), jnp.float32),
        scratch_shapes=[
            pltpu.VMEM((_B, _DA), jnp.float32),
            pltpu.VMEM((_B, _BLK), jnp.bfloat16),
            pltpu.VMEM((_BLK, _DA), jnp.bfloat16),
        ],
    )(x, batch3, w1t, w2t)


# speedup vs baseline: 1.0775x; 1.0775x over previous
import jax
import jax.numpy as jnp
from jax.experimental import pallas as pl
from jax.experimental.pallas import tpu as pltpu

_N, _D, _H, _B = 16384, 512, 256, 16
_BLK = 4096
_NB = _N // _BLK


def _pool_body(xb, bb, w1t, w2t, out, acc, sstate, mprev, aprev):
    i = pl.program_id(0)

    @pl.when(i == 0)
    def _init():
        acc[:] = jnp.zeros_like(acc)
        sstate[:] = jnp.zeros_like(sstate)

    x_bf = xb[:].astype(jnp.bfloat16)
    t_bf = jnp.tanh(jnp.dot(x_bf, w1t[:],
                            preferred_element_type=jnp.float32)
                    ).astype(jnp.bfloat16)
    a = jnp.dot(t_bf, w2t[:],
                preferred_element_type=jnp.float32)
    e_row = jnp.exp(a.reshape(1, _BLK))
    b = bb[0]
    seg = jax.lax.broadcasted_iota(jnp.int32, (_B, _BLK), 0)
    w_mat = jnp.where(b == seg, e_row, 0.0)
    sstate[:] = sstate[:] + jnp.sum(w_mat, axis=1, keepdims=True)

    @pl.when(i > 0)
    def _pool_prev():
        acc[:] = acc[:] + jax.lax.dot_general(
            mprev[:], aprev[:], (((1,), (0,)), ((), ())),
            preferred_element_type=jnp.float32)

    mprev[:] = w_mat.astype(jnp.bfloat16)
    aprev[:] = x_bf

    @pl.when(i == _NB - 1)
    def _fin():
        z = acc[:] + jax.lax.dot_general(
            mprev[:], aprev[:], (((1,), (0,)), ((), ())),
            preferred_element_type=jnp.float32)
        out[:] = z / (sstate[:] + 1e-16)


def kernel(x, batch, W1, W2):
    batch3 = batch.astype(jnp.int32).reshape(_NB, 1, _BLK)
    w1t = W1.T.astype(jnp.bfloat16)
    w2t = W2.T.astype(jnp.bfloat16)
    return pl.pallas_call(
        _pool_body,
        grid=(_NB,),
        in_specs=[
            pl.BlockSpec((_BLK, _D), lambda i: (i, 0)),
            pl.BlockSpec((1, 1, _BLK), lambda i: (i, 0, 0)),
            pl.BlockSpec((_D, _H), lambda i: (0, 0)),
            pl.BlockSpec((_H, 1), lambda i: (0, 0)),
        ],
        out_specs=pl.BlockSpec((_B, _D), lambda i: (0, 0)),
        out_shape=jax.ShapeDtypeStruct((_B, _D), jnp.float32),
        scratch_shapes=[
            pltpu.VMEM((_B, _D), jnp.float32),
            pltpu.VMEM((_B, 1), jnp.float32),
            pltpu.VMEM((_B, _BLK), jnp.bfloat16),
            pltpu.VMEM((_BLK, _D), jnp.bfloat16),
        ],
    )(x, batch3, w1t, w2t)


# speedup vs baseline: 1.1467x; 1.0642x over previous
import jax
import jax.numpy as jnp
from jax.experimental import pallas as pl
from jax.experimental.pallas import tpu as pltpu

_N, _D, _H, _B = 16384, 512, 256, 16
_BLK = 4096
_NB = _N // _BLK


def _pool_body(xb, bb, w1t, w2t, out, acc, sstate, mprev, aprev):
    i = pl.program_id(0)

    @pl.when(i == 0)
    def _init():
        acc[:] = jnp.zeros_like(acc)
        sstate[:] = jnp.zeros_like(sstate)

    @pl.when(i > 0)
    def _pool_prev():
        acc[:] = acc[:] + jax.lax.dot_general(
            mprev[:], aprev[:], (((1,), (0,)), ((), ())),
            preferred_element_type=jnp.float32)

    aprev[:] = xb[:].astype(jnp.bfloat16)
    t_bf = jnp.tanh(jnp.dot(aprev[:], w1t[:],
                            preferred_element_type=jnp.float32)
                    ).astype(jnp.bfloat16)
    a = jnp.dot(t_bf, w2t[:],
                preferred_element_type=jnp.float32)
    e_row = jnp.exp(a.reshape(1, _BLK))
    b = bb[0]
    seg = jax.lax.broadcasted_iota(jnp.int32, (_B, _BLK), 0)
    w_mat = jnp.where(b == seg, e_row, 0.0)
    sstate[:] = sstate[:] + jnp.sum(w_mat, axis=1, keepdims=True)

    mprev[:] = w_mat.astype(jnp.bfloat16)

    @pl.when(i == _NB - 1)
    def _fin():
        z = acc[:] + jax.lax.dot_general(
            mprev[:], aprev[:], (((1,), (0,)), ((), ())),
            preferred_element_type=jnp.float32)
        out[:] = z / (sstate[:] + 1e-16)


def kernel(x, batch, W1, W2):
    batch3 = batch.astype(jnp.int32).reshape(_NB, 1, _BLK)
    w1t = W1.T.astype(jnp.bfloat16)
    w2t = W2.T.astype(jnp.bfloat16)
    return pl.pallas_call(
        _pool_body,
        grid=(_NB,),
        in_specs=[
            pl.BlockSpec((_BLK, _D), lambda i: (i, 0)),
            pl.BlockSpec((1, 1, _BLK), lambda i: (i, 0, 0)),
            pl.BlockSpec((_D, _H), lambda i: (0, 0)),
            pl.BlockSpec((_H, 1), lambda i: (0, 0)),
        ],
        out_specs=pl.BlockSpec((_B, _D), lambda i: (0, 0)),
        out_shape=jax.ShapeDtypeStruct((_B, _D), jnp.float32),
        scratch_shapes=[
            pltpu.VMEM((_B, _D), jnp.float32),
            pltpu.VMEM((_B, 1), jnp.float32),
            pltpu.VMEM((_B, _BLK), jnp.bfloat16),
            pltpu.VMEM((_BLK, _D), jnp.bfloat16),
        ],
    )(x, batch3, w1t, w2t)


# speedup vs baseline: 1.1500x; 1.0029x over previous
import jax
import jax.numpy as jnp
from jax.experimental import pallas as pl
from jax.experimental.pallas import tpu as pltpu

_N, _D, _H, _B = 16384, 512, 256, 16
_BLK = 4096
_NB = _N // _BLK


def _pool_body(xb, bb, w1t, w2t, out, acc, sstate, mprev, aprev):
    i = pl.program_id(0)

    @pl.when(i == 0)
    def _init():
        acc[:] = jnp.zeros_like(acc)
        sstate[:] = jnp.zeros_like(sstate)

    cur = jax.lax.rem(i, 2)
    prv = 1 - cur

    @pl.when(i > 0)
    def _pool_prev():
        acc[:] = acc[:] + jax.lax.dot_general(
            mprev[prv], aprev[prv], (((1,), (0,)), ((), ())),
            preferred_element_type=jnp.float32)

    aprev[cur] = xb[:].astype(jnp.bfloat16)
    t_bf = jnp.tanh(jnp.dot(aprev[cur], w1t[:],
                            preferred_element_type=jnp.float32)
                    ).astype(jnp.bfloat16)
    a = jnp.dot(t_bf, w2t[:],
                preferred_element_type=jnp.float32)
    e_row = jnp.exp(a.reshape(1, _BLK))
    b = bb[0]
    seg = jax.lax.broadcasted_iota(jnp.int32, (_B, _BLK), 0)
    w_mat = jnp.where(b == seg, e_row, 0.0)
    sstate[:] = sstate[:] + jnp.sum(w_mat, axis=1, keepdims=True)

    mprev[cur] = w_mat.astype(jnp.bfloat16)

    @pl.when(i == _NB - 1)
    def _fin():
        z = acc[:] + jax.lax.dot_general(
            mprev[cur], aprev[cur], (((1,), (0,)), ((), ())),
            preferred_element_type=jnp.float32)
        out[:] = z / (sstate[:] + 1e-16)


def kernel(x, batch, W1, W2):
    batch3 = batch.astype(jnp.int32).reshape(_NB, 1, _BLK)
    w1t = W1.T.astype(jnp.bfloat16)
    w2t = W2.T.astype(jnp.bfloat16)
    return pl.pallas_call(
        _pool_body,
        grid=(_NB,),
        in_specs=[
            pl.BlockSpec((_BLK, _D), lambda i: (i, 0)),
            pl.BlockSpec((1, 1, _BLK), lambda i: (i, 0, 0)),
            pl.BlockSpec((_D, _H), lambda i: (0, 0)),
            pl.BlockSpec((_H, 1), lambda i: (0, 0)),
        ],
        out_specs=pl.BlockSpec((_B, _D), lambda i: (0, 0)),
        out_shape=jax.ShapeDtypeStruct((_B, _D), jnp.float32),
        scratch_shapes=[
            pltpu.VMEM((_B, _D), jnp.float32),
            pltpu.VMEM((_B, 1), jnp.float32),
            pltpu.VMEM((2, _B, _BLK), jnp.bfloat16),
            pltpu.VMEM((2, _BLK, _D), jnp.bfloat16),
        ],
    )(x, batch3, w1t, w2t)
